# baseline (speedup 1.0000x reference)
import functools

import jax
import jax.numpy as jnp
from jax import lax
from jax.experimental import pallas as pl
from jax.experimental.pallas import tpu as pltpu
from jax.experimental.pallas import tpu_sc as plsc

K = 20
R = 256
NEG_INF = float("-inf")

SC_NC = 2
SC_NS = 16
SC_NW = SC_NC * SC_NS


def _sc_gather_rows(table, idxg, n_rows):
    C = table.shape[-1]
    nq = n_rows // (SC_NW * 512)
    mesh = plsc.VectorSubcoreMesh(
        core_axis_name="c", subcore_axis_name="s",
        num_cores=SC_NC, num_subcores=SC_NS)

    @functools.partial(
        pl.kernel,
        out_type=jax.ShapeDtypeStruct((n_rows, C), jnp.float32),
        mesh=mesh,
        scratch_types=[
            pltpu.VMEM((512,), jnp.int32),
            pltpu.VMEM((512, C), jnp.float32),
            pltpu.SemaphoreType.DMA,
        ],
    )
    def gather_kernel(table_hbm, idx_hbm, out_hbm, idx_v, buf, sem):
        wid = lax.axis_index("s") * SC_NC + lax.axis_index("c")
        for q in range(nq):
            off = pl.multiple_of((wid * nq + q) * 512, 512)
            pltpu.sync_copy(idx_hbm.at[pl.ds(off, 512)], idx_v)
            pltpu.async_copy(table_hbm.at[idx_v], buf, sem).wait()
            pltpu.sync_copy(buf, out_hbm.at[pl.ds(off, 512)])

    return gather_kernel(table, idxg)


def _leaky(v):
    return jnp.where(v >= 0, v, 0.2 * v)


def _neg_dist(xr, xb, sqr, sqb):
    g = lax.dot_general(
        xr, xb, (((1,), (1,)), ((), ())),
        preferred_element_type=jnp.float32,
    )
    return -((sqr - 2.0 * g) + sqb)


def _topk_loop_t(nd_t):
    n, rr = nd_t.shape
    siota = lax.broadcasted_iota(jnp.int32, (n, rr), 0)
    kiota = lax.broadcasted_iota(jnp.int32, (K, rr), 0)

    def body(j, carry):
        nd, idxmat = carry
        m = jnp.max(nd, axis=0, keepdims=True)
        am = jnp.min(jnp.where(nd == m, siota, n),
                     axis=0, keepdims=True)
        nd = jnp.where(siota == am, NEG_INF, nd)
        idxmat = idxmat + jnp.where(kiota == j, am, 0)
        return nd, idxmat

    _, idxmat = lax.fori_loop(
        0, K, body, (nd_t, jnp.zeros((K, rr), jnp.int32)))
    return idxmat


def _topk_threshold(nd_t):
    rr = nd_t.shape[1]

    def body(_, carry):
        prev_m, cnt = carry
        masked = jnp.where(nd_t < prev_m, nd_t, NEG_INF)
        m_new = jnp.max(masked, axis=0, keepdims=True)
        cnt_new = jnp.sum((nd_t >= m_new).astype(jnp.float32),
                          axis=0, keepdims=True)
        upd = cnt < float(K)
        prev_m = jnp.where(upd, m_new, prev_m)
        cnt = jnp.where(upd, cnt_new, cnt)
        return prev_m, cnt

    tau, _ = lax.fori_loop(
        0, K, body, (jnp.full((1, rr), jnp.inf, jnp.float32),
                     jnp.zeros((1, rr), jnp.float32)))
    return tau


def _knn_kernel(xf_ref, xr_ref, sqrow_ref, sqcand_ref, idx_ref):
    g = lax.dot_general(
        xf_ref[0], xr_ref[0], (((1,), (1,)), ((), ())),
        preferred_element_type=jnp.float32)
    nd_t = -((sqrow_ref[0] - 2.0 * g) + sqcand_ref[0])
    idx_ref[0] = _topk_loop_t(nd_t)


def _edge_kernel(xr_ref, xj_ref, w1_ref, b1_ref, w2_ref, b2_ref, x1_ref):
    xr = xr_ref[0]
    rr, c = xr.shape
    xj = xj_ref[0][:, :, :c]
    xi = jnp.broadcast_to(xr[:, None, :], (rr, K, c))
    e = jnp.concatenate([xi, xj - xi], axis=2)
    ef = e.reshape(rr * K, 2 * c)
    h = _leaky(lax.dot(ef, w1_ref[...],
                       preferred_element_type=jnp.float32) + b1_ref[...])
    h = _leaky(lax.dot(h, w2_ref[...],
                       preferred_element_type=jnp.float32) + b2_ref[...])
    x1_ref[0] = jnp.max(h.reshape(rr, K, -1), axis=1)


def _idgcn_kernel(xf_ref, xr_ref, sqrow_ref, sqcand_ref, ws_ref, wn_ref,
                  bb_ref, xo_ref):
    xb = xf_ref[0]
    xr = xr_ref[0]
    g = lax.dot_general(
        xb, xr, (((1,), (1,)), ((), ())),
        preferred_element_type=jnp.float32)
    nd_t = -((sqrow_ref[0] - 2.0 * g) + sqcand_ref[0])
    tau = _topk_threshold(nd_t)
    acc = (nd_t >= tau).astype(jnp.float32)
    agg = lax.dot_general(
        acc, xb, (((0,), (0,)), ((), ())),
        preferred_element_type=jnp.float32,
        precision=lax.Precision.HIGHEST) / jnp.float32(K)
    h = _leaky((lax.dot(xr, ws_ref[...], preferred_element_type=jnp.float32)
                + lax.dot(agg, wn_ref[...],
                          preferred_element_type=jnp.float32))
               + bb_ref[...])
    xo_ref[0] = xr + h


def _full_spec(n, c):
    return pl.BlockSpec((1, n, c), lambda b, rb: (b, 0, 0))


def _row_spec(c):
    return pl.BlockSpec((1, R, c), lambda b, rb: (b, rb, 0))


def _w_spec(h, w):
    return pl.BlockSpec((h, w), lambda b, rb: (0, 0))


@jax.jit
def kernel(feature, W1, b1, W2, b2, Ws1, Wn1, bb1, Ws2, Wn2, bb2,
           Ws3, Wn3, bb3):
    B, N, C = feature.shape
    H = W2.shape[0]
    nb = N // R
    cparams = pltpu.CompilerParams(
        dimension_semantics=("parallel", "arbitrary"))
    sqrow_spec = pl.BlockSpec((1, 1, R), lambda b, rb: (b, 0, rb))
    sqcand_spec = pl.BlockSpec((1, N, 1), lambda b, rb: (b, 0, 0))

    sq = jnp.sum(feature * feature, axis=-1)
    idx_t = pl.pallas_call(
        _knn_kernel,
        grid=(B, nb),
        in_specs=[_full_spec(N, C), _row_spec(C), sqrow_spec, sqcand_spec],
        out_specs=pl.BlockSpec((1, K, R), lambda b, rb: (b, 0, rb)),
        out_shape=jax.ShapeDtypeStruct((B, K, N), jnp.int32),
        compiler_params=cparams,
    )(feature, feature, sq.reshape(B, 1, N), sq.reshape(B, N, 1))
    idx = jnp.transpose(idx_t, (0, 2, 1))

    idxg = (idx + (jnp.arange(B, dtype=jnp.int32) * N)[:, None, None])
    idxg = idxg.reshape(B * N * K)
    table = jnp.concatenate(
        [feature.reshape(B * N, C),
         jnp.zeros((B * N, 128 - C), jnp.float32)], axis=1)
    xj = _sc_gather_rows(table, idxg, B * N * K)
    xj = xj.reshape(B, N, K, 128)

    x = pl.pallas_call(
        _edge_kernel,
        grid=(B, nb),
        in_specs=[_row_spec(C),
                  pl.BlockSpec((1, R, K, 128), lambda b, rb: (b, rb, 0, 0)),
                  _w_spec(2 * C, H), _w_spec(1, H), _w_spec(H, H),
                  _w_spec(1, H)],
        out_specs=_row_spec(H),
        out_shape=jax.ShapeDtypeStruct((B, N, H), jnp.float32),
        compiler_params=cparams,
    )(feature, xj, W1, b1.reshape(1, H), W2, b2.reshape(1, H))

    feats = []
    for (Ws, Wn, bb) in ((Ws1, Wn1, bb1), (Ws2, Wn2, bb2), (Ws3, Wn3, bb3)):
        sq = jnp.sum(x * x, axis=-1)
        x = pl.pallas_call(
            _idgcn_kernel,
            grid=(B, nb),
            in_specs=[_full_spec(N, H), _row_spec(H), sqrow_spec,
                      sqcand_spec,
                      _w_spec(H, H), _w_spec(H, H), _w_spec(1, H)],
            out_specs=_row_spec(H),
            out_shape=jax.ShapeDtypeStruct((B, N, H), jnp.float32),
            compiler_params=cparams,
        )(x, x, sq.reshape(B, 1, N), sq.reshape(B, N, 1),
          Ws, Wn, bb.reshape(1, H))
        feats.append(x)

    out = jnp.concatenate(feats, axis=-1)
    return jnp.transpose(out, (0, 2, 1))


# speedup vs baseline: 1.2896x; 1.2896x over previous
import functools

import jax
import jax.numpy as jnp
from jax import lax
from jax.experimental import pallas as pl
from jax.experimental.pallas import tpu as pltpu
from jax.experimental.pallas import tpu_sc as plsc

K = 20
R = 256
NEG_INF = float("-inf")

SC_NC = 2
SC_NS = 16
SC_NW = SC_NC * SC_NS


def _sc_gather_rows(table, idxg, n_rows):
    C = table.shape[-1]
    nq = n_rows // (SC_NW * 512)
    mesh = plsc.VectorSubcoreMesh(
        core_axis_name="c", subcore_axis_name="s",
        num_cores=SC_NC, num_subcores=SC_NS)

    @functools.partial(
        pl.kernel,
        out_type=jax.ShapeDtypeStruct((n_rows, C), jnp.float32),
        mesh=mesh,
        scratch_types=[
            pltpu.VMEM((512,), jnp.int32),
            pltpu.VMEM((512, C), jnp.float32),
            pltpu.SemaphoreType.DMA,
        ],
    )
    def gather_kernel(table_hbm, idx_hbm, out_hbm, idx_v, buf, sem):
        wid = lax.axis_index("s") * SC_NC + lax.axis_index("c")
        for q in range(nq):
            off = pl.multiple_of((wid * nq + q) * 512, 512)
            pltpu.sync_copy(idx_hbm.at[pl.ds(off, 512)], idx_v)
            pltpu.async_copy(table_hbm.at[idx_v], buf, sem).wait()
            pltpu.sync_copy(buf, out_hbm.at[pl.ds(off, 512)])

    return gather_kernel(table, idxg)


def _leaky(v):
    return jnp.where(v >= 0, v, 0.2 * v)


def _neg_dist(xr, xb, sqr, sqb):
    g = lax.dot_general(
        xr, xb, (((1,), (1,)), ((), ())),
        preferred_element_type=jnp.float32,
    )
    return -((sqr - 2.0 * g) + sqb)


def _topk_loop_t(nd_t):
    n, rr = nd_t.shape
    siota = lax.broadcasted_iota(jnp.int32, (n, rr), 0)
    kiota = lax.broadcasted_iota(jnp.int32, (K, rr), 0)

    def body(j, carry):
        nd, idxmat = carry
        m = jnp.max(nd, axis=0, keepdims=True)
        am = jnp.min(jnp.where(nd == m, siota, n),
                     axis=0, keepdims=True)
        nd = jnp.where(siota == am, NEG_INF, nd)
        idxmat = idxmat + jnp.where(kiota == j, am, 0)
        return nd, idxmat

    _, idxmat = lax.fori_loop(
        0, K, body, (nd_t, jnp.zeros((K, rr), jnp.int32)))
    return idxmat


def _topk_threshold(nd_t):
    rr = nd_t.shape[1]

    def body(_, prev_m):
        masked = jnp.where(nd_t < prev_m, nd_t, NEG_INF)
        return jnp.max(masked, axis=0, keepdims=True)

    return lax.fori_loop(
        0, K, body, jnp.full((1, rr), jnp.inf, jnp.float32))


def _knn_kernel(xf_ref, xr_ref, sqrow_ref, sqcand_ref, idx_ref):
    g = lax.dot_general(
        xf_ref[0], xr_ref[0], (((1,), (1,)), ((), ())),
        preferred_element_type=jnp.float32)
    nd_t = -((sqrow_ref[0] - 2.0 * g) + sqcand_ref[0])
    idx_ref[0] = _topk_loop_t(nd_t)


def _edge_kernel(xr_ref, xj_ref, w1_ref, b1_ref, w2_ref, b2_ref, x1_ref):
    xr = xr_ref[0]
    rr, c = xr.shape
    xj = xj_ref[0][:, :, :c]
    xi = jnp.broadcast_to(xr[:, None, :], (rr, K, c))
    e = jnp.concatenate([xi, xj - xi], axis=2)
    ef = e.reshape(rr * K, 2 * c)
    h = _leaky(lax.dot(ef, w1_ref[...],
                       preferred_element_type=jnp.float32) + b1_ref[...])
    h = _leaky(lax.dot(h, w2_ref[...],
                       preferred_element_type=jnp.float32) + b2_ref[...])
    x1_ref[0] = jnp.max(h.reshape(rr, K, -1), axis=1)


def _idgcn_kernel(xf_ref, xr_ref, sqrow_ref, sqcand_ref, ws_ref, wn_ref,
                  bb_ref, xo_ref):
    xb = xf_ref[0]
    xr = xr_ref[0]
    g = lax.dot_general(
        xb, xr, (((1,), (1,)), ((), ())),
        preferred_element_type=jnp.float32)
    nd_t = -((sqrow_ref[0] - 2.0 * g) + sqcand_ref[0])
    tau = _topk_threshold(nd_t)
    acc = (nd_t >= tau).astype(jnp.float32)
    agg = lax.dot_general(
        acc, xb, (((0,), (0,)), ((), ())),
        preferred_element_type=jnp.float32,
        precision=lax.Precision.HIGHEST) / jnp.float32(K)
    h = _leaky((lax.dot(xr, ws_ref[...], preferred_element_type=jnp.float32)
                + lax.dot(agg, wn_ref[...],
                          preferred_element_type=jnp.float32))
               + bb_ref[...])
    xo_ref[0] = xr + h


def _full_spec(n, c):
    return pl.BlockSpec((1, n, c), lambda b, rb: (b, 0, 0))


def _row_spec(c):
    return pl.BlockSpec((1, R, c), lambda b, rb: (b, rb, 0))


def _w_spec(h, w):
    return pl.BlockSpec((h, w), lambda b, rb: (0, 0))


@jax.jit
def kernel(feature, W1, b1, W2, b2, Ws1, Wn1, bb1, Ws2, Wn2, bb2,
           Ws3, Wn3, bb3):
    B, N, C = feature.shape
    H = W2.shape[0]
    nb = N // R
    cparams = pltpu.CompilerParams(
        dimension_semantics=("parallel", "arbitrary"))
    sqrow_spec = pl.BlockSpec((1, 1, R), lambda b, rb: (b, 0, rb))
    sqcand_spec = pl.BlockSpec((1, N, 1), lambda b, rb: (b, 0, 0))

    sq = jnp.sum(feature * feature, axis=-1)
    idx_t = pl.pallas_call(
        _knn_kernel,
        grid=(B, nb),
        in_specs=[_full_spec(N, C), _row_spec(C), sqrow_spec, sqcand_spec],
        out_specs=pl.BlockSpec((1, K, R), lambda b, rb: (b, 0, rb)),
        out_shape=jax.ShapeDtypeStruct((B, K, N), jnp.int32),
        compiler_params=cparams,
    )(feature, feature, sq.reshape(B, 1, N), sq.reshape(B, N, 1))
    idx = jnp.transpose(idx_t, (0, 2, 1))

    idxg = (idx + (jnp.arange(B, dtype=jnp.int32) * N)[:, None, None])
    idxg = idxg.reshape(B * N * K)
    table = jnp.concatenate(
        [feature.reshape(B * N, C),
         jnp.zeros((B * N, 128 - C), jnp.float32)], axis=1)
    xj = _sc_gather_rows(table, idxg, B * N * K)
    xj = xj.reshape(B, N, K, 128)

    x = pl.pallas_call(
        _edge_kernel,
        grid=(B, nb),
        in_specs=[_row_spec(C),
                  pl.BlockSpec((1, R, K, 128), lambda b, rb: (b, rb, 0, 0)),
                  _w_spec(2 * C, H), _w_spec(1, H), _w_spec(H, H),
                  _w_spec(1, H)],
        out_specs=_row_spec(H),
        out_shape=jax.ShapeDtypeStruct((B, N, H), jnp.float32),
        compiler_params=cparams,
    )(feature, xj, W1, b1.reshape(1, H), W2, b2.reshape(1, H))

    feats = []
    for (Ws, Wn, bb) in ((Ws1, Wn1, bb1), (Ws2, Wn2, bb2), (Ws3, Wn3, bb3)):
        sq = jnp.sum(x * x, axis=-1)
        x = pl.pallas_call(
            _idgcn_kernel,
            grid=(B, nb),
            in_specs=[_full_spec(N, H), _row_spec(H), sqrow_spec,
                      sqcand_spec,
                      _w_spec(H, H), _w_spec(H, H), _w_spec(1, H)],
            out_specs=_row_spec(H),
            out_shape=jax.ShapeDtypeStruct((B, N, H), jnp.float32),
            compiler_params=cparams,
        )(x, x, sq.reshape(B, 1, N), sq.reshape(B, N, 1),
          Ws, Wn, bb.reshape(1, H))
        feats.append(x)

    out = jnp.concatenate(feats, axis=-1)
    return jnp.transpose(out, (0, 2, 1))


# speedup vs baseline: 1.3044x; 1.0115x over previous
import functools

import jax
import jax.numpy as jnp
from jax import lax
from jax.experimental import pallas as pl
from jax.experimental.pallas import tpu as pltpu
from jax.experimental.pallas import tpu_sc as plsc

K = 20
R = 256
NEG_INF = float("-inf")

SC_NC = 2
SC_NS = 16
SC_NW = SC_NC * SC_NS


def _sc_gather_rows(table, idxg, n_rows):
    C = table.shape[-1]
    nq = n_rows // (SC_NW * 512)
    mesh = plsc.VectorSubcoreMesh(
        core_axis_name="c", subcore_axis_name="s",
        num_cores=SC_NC, num_subcores=SC_NS)

    @functools.partial(
        pl.kernel,
        out_type=jax.ShapeDtypeStruct((n_rows, C), jnp.float32),
        mesh=mesh,
        scratch_types=[
            pltpu.VMEM((512,), jnp.int32),
            pltpu.VMEM((512, C), jnp.float32),
            pltpu.SemaphoreType.DMA,
        ],
    )
    def gather_kernel(table_hbm, idx_hbm, out_hbm, idx_v, buf, sem):
        wid = lax.axis_index("s") * SC_NC + lax.axis_index("c")
        for q in range(nq):
            off = pl.multiple_of((wid * nq + q) * 512, 512)
            pltpu.sync_copy(idx_hbm.at[pl.ds(off, 512)], idx_v)
            pltpu.async_copy(table_hbm.at[idx_v], buf, sem).wait()
            pltpu.sync_copy(buf, out_hbm.at[pl.ds(off, 512)])

    return gather_kernel(table, idxg)


def _leaky(v):
    return jnp.where(v >= 0, v, 0.2 * v)


def _neg_dist(xr, xb, sqr, sqb):
    g = lax.dot_general(
        xr, xb, (((1,), (1,)), ((), ())),
        preferred_element_type=jnp.float32,
    )
    return -((sqr - 2.0 * g) + sqb)


def _topk_loop_t(nd_t):
    n, rr = nd_t.shape
    siota = lax.broadcasted_iota(jnp.int32, (n, rr), 0)
    kiota = lax.broadcasted_iota(jnp.int32, (K, rr), 0)

    def body(j, carry):
        vprev, iprev, idxmat = carry
        elig = jnp.logical_or(
            nd_t < vprev,
            jnp.logical_and(nd_t == vprev, siota > iprev))
        m = jnp.max(jnp.where(elig, nd_t, NEG_INF),
                    axis=0, keepdims=True)
        am = jnp.min(
            jnp.where(jnp.logical_and(elig, nd_t == m), siota, n),
            axis=0, keepdims=True)
        idxmat = idxmat + jnp.where(kiota == j, am, 0)
        return m, am, idxmat

    _, _, idxmat = lax.fori_loop(
        0, K, body,
        (jnp.full((1, rr), jnp.inf, jnp.float32),
         jnp.full((1, rr), -1, jnp.int32),
         jnp.zeros((K, rr), jnp.int32)))
    return idxmat


def _topk_threshold(nd_t):
    rr = nd_t.shape[1]

    def body(_, prev_m):
        masked = jnp.where(nd_t < prev_m, nd_t, NEG_INF)
        return jnp.max(masked, axis=0, keepdims=True)

    return lax.fori_loop(
        0, K, body, jnp.full((1, rr), jnp.inf, jnp.float32))


def _knn_kernel(xf_ref, xr_ref, sqrow_ref, sqcand_ref, idx_ref):
    g = lax.dot_general(
        xf_ref[0], xr_ref[0], (((1,), (1,)), ((), ())),
        preferred_element_type=jnp.float32)
    nd_t = -((sqrow_ref[0] - 2.0 * g) + sqcand_ref[0])
    idx_ref[0] = _topk_loop_t(nd_t)


def _edge_kernel(xr_ref, xj_ref, w1_ref, b1_ref, w2_ref, b2_ref, x1_ref):
    xr = xr_ref[0]
    rr, c = xr.shape
    xj = xj_ref[0][:, :, :c]
    xi = jnp.broadcast_to(xr[:, None, :], (rr, K, c))
    e = jnp.concatenate([xi, xj - xi], axis=2)
    ef = e.reshape(rr * K, 2 * c)
    h = _leaky(lax.dot(ef, w1_ref[...],
                       preferred_element_type=jnp.float32) + b1_ref[...])
    h = _leaky(lax.dot(h, w2_ref[...],
                       preferred_element_type=jnp.float32) + b2_ref[...])
    x1_ref[0] = jnp.max(h.reshape(rr, K, -1), axis=1)


def _idgcn_kernel(xf_ref, xr_ref, sqrow_ref, sqcand_ref, ws_ref, wn_ref,
                  bb_ref, xo_ref):
    xb = xf_ref[0]
    xr = xr_ref[0]
    g = lax.dot_general(
        xb, xr, (((1,), (1,)), ((), ())),
        preferred_element_type=jnp.float32)
    nd_t = -((sqrow_ref[0] - 2.0 * g) + sqcand_ref[0])
    tau = _topk_threshold(nd_t)
    acc = (nd_t >= tau).astype(jnp.float32)
    agg = lax.dot_general(
        acc, xb, (((0,), (0,)), ((), ())),
        preferred_element_type=jnp.float32,
        precision=lax.Precision.HIGHEST) / jnp.float32(K)
    h = _leaky((lax.dot(xr, ws_ref[...], preferred_element_type=jnp.float32)
                + lax.dot(agg, wn_ref[...],
                          preferred_element_type=jnp.float32))
               + bb_ref[...])
    xo_ref[0] = xr + h


def _full_spec(n, c):
    return pl.BlockSpec((1, n, c), lambda b, rb: (b, 0, 0))


def _row_spec(c):
    return pl.BlockSpec((1, R, c), lambda b, rb: (b, rb, 0))


def _w_spec(h, w):
    return pl.BlockSpec((h, w), lambda b, rb: (0, 0))


@jax.jit
def kernel(feature, W1, b1, W2, b2, Ws1, Wn1, bb1, Ws2, Wn2, bb2,
           Ws3, Wn3, bb3):
    B, N, C = feature.shape
    H = W2.shape[0]
    nb = N // R
    cparams = pltpu.CompilerParams(
        dimension_semantics=("parallel", "arbitrary"))
    sqrow_spec = pl.BlockSpec((1, 1, R), lambda b, rb: (b, 0, rb))
    sqcand_spec = pl.BlockSpec((1, N, 1), lambda b, rb: (b, 0, 0))

    sq = jnp.sum(feature * feature, axis=-1)
    idx_t = pl.pallas_call(
        _knn_kernel,
        grid=(B, nb),
        in_specs=[_full_spec(N, C), _row_spec(C), sqrow_spec, sqcand_spec],
        out_specs=pl.BlockSpec((1, K, R), lambda b, rb: (b, 0, rb)),
        out_shape=jax.ShapeDtypeStruct((B, K, N), jnp.int32),
        compiler_params=cparams,
    )(feature, feature, sq.reshape(B, 1, N), sq.reshape(B, N, 1))
    idx = jnp.transpose(idx_t, (0, 2, 1))

    idxg = (idx + (jnp.arange(B, dtype=jnp.int32) * N)[:, None, None])
    idxg = idxg.reshape(B * N * K)
    table = jnp.concatenate(
        [feature.reshape(B * N, C),
         jnp.zeros((B * N, 128 - C), jnp.float32)], axis=1)
    xj = _sc_gather_rows(table, idxg, B * N * K)
    xj = xj.reshape(B, N, K, 128)

    x = pl.pallas_call(
        _edge_kernel,
        grid=(B, nb),
        in_specs=[_row_spec(C),
                  pl.BlockSpec((1, R, K, 128), lambda b, rb: (b, rb, 0, 0)),
                  _w_spec(2 * C, H), _w_spec(1, H), _w_spec(H, H),
                  _w_spec(1, H)],
        out_specs=_row_spec(H),
        out_shape=jax.ShapeDtypeStruct((B, N, H), jnp.float32),
        compiler_params=cparams,
    )(feature, xj, W1, b1.reshape(1, H), W2, b2.reshape(1, H))

    feats = []
    for (Ws, Wn, bb) in ((Ws1, Wn1, bb1), (Ws2, Wn2, bb2), (Ws3, Wn3, bb3)):
        sq = jnp.sum(x * x, axis=-1)
        x = pl.pallas_call(
            _idgcn_kernel,
            grid=(B, nb),
            in_specs=[_full_spec(N, H), _row_spec(H), sqrow_spec,
                      sqcand_spec,
                      _w_spec(H, H), _w_spec(H, H), _w_spec(1, H)],
            out_specs=_row_spec(H),
            out_shape=jax.ShapeDtypeStruct((B, N, H), jnp.float32),
            compiler_params=cparams,
        )(x, x, sq.reshape(B, 1, N), sq.reshape(B, N, 1),
          Ws, Wn, bb.reshape(1, H))
        feats.append(x)

    out = jnp.concatenate(feats, axis=-1)
    return jnp.transpose(out, (0, 2, 1))
